# baseline (speedup 1.0000x reference)
import functools

import jax
import jax.numpy as jnp
from jax import lax
from jax.experimental import pallas as pl
from jax.experimental.pallas import tpu as pltpu
from jax.experimental.pallas import tpu_sc as plsc

N = 10000
E = 320000
D = 128

NC = 2
NS = 16
NW = NC * NS
EPT = E // NW
K = 128
KD = 128
NCHD = 80
KS = 80
NCHS = 128
EPAD = 10240
NPAD = 10240
WPT = NPAD // NS
RPT = NPAD // NS

_mesh = plsc.VectorSubcoreMesh(
    core_axis_name="c", subcore_axis_name="s", num_cores=NC, num_subcores=NS)


@functools.partial(
    pl.kernel,
    out_type=jax.ShapeDtypeStruct((NW, WPT), jnp.float32),
    mesh=_mesh,
    scratch_types=dict(
        deg_sh=pltpu.VMEM_SHARED((NPAD,), jnp.float32),
        zbuf=pltpu.VMEM((WPT,), jnp.float32),
        ones_v=pltpu.VMEM((KD,), jnp.float32),
        didx2=pltpu.VMEM((NCHD, KD), jnp.int32),
    ),
)
def _deg_kernel(dst_hbm, out_hbm, deg_sh, zbuf, ones_v, didx2):
    cid = lax.axis_index("c")
    sid = lax.axis_index("s")
    wid = sid * NC + cid

    one = jnp.ones((16,), jnp.float32)
    zero = jnp.zeros((16,), jnp.float32)

    def fill(i, _):
        zbuf[pl.ds(i * 16, 16)] = zero
        ones_v[pl.ds((i % (KD // 16)) * 16, 16)] = one
        return 0

    lax.fori_loop(0, WPT // 16, fill, 0)

    pltpu.sync_copy(dst_hbm.at[wid], didx2)
    pltpu.sync_copy(zbuf, deg_sh.at[pl.ds(sid * WPT, WPT)])
    plsc.subcore_barrier()

    def step(i, _):
        pltpu.sync_copy(ones_v, deg_sh.at[didx2.at[i]], add=True)
        return 0

    lax.fori_loop(0, NCHD, step, 0)
    plsc.subcore_barrier()

    pltpu.sync_copy(deg_sh.at[pl.ds(sid * WPT, WPT)],
                    out_hbm.at[cid * NS + sid])


@functools.partial(
    pl.kernel,
    out_type=jax.ShapeDtypeStruct((NC, NPAD, D), jnp.float32),
    mesh=_mesh,
    scratch_types=dict(
        acc_sh=pltpu.VMEM_SHARED((NPAD, D), jnp.float32),
        rows0=pltpu.VMEM((KS, D), jnp.float32),
        sidxA=pltpu.VMEM((KS,), jnp.int32),
        didxA=pltpu.VMEM((KS,), jnp.int32),
        sidxB=pltpu.VMEM((KS,), jnp.int32),
        didxB=pltpu.VMEM((KS,), jnp.int32),
        gsem0=pltpu.SemaphoreType.DMA,
        isem=pltpu.SemaphoreType.DMA,
    ),
)
def _scatter_kernel(xs_hbm, src_hbm, dst_hbm, out_hbm,
                    acc_sh, rows0, sidxA, didxA, sidxB, didxB, gsem0, isem):
    cid = lax.axis_index("c")
    sid = lax.axis_index("s")
    wid = sid * NC + cid

    zero = jnp.zeros((16,), jnp.float32)

    def zfill(i, _):
        r = i // 8
        c = i - r * 8
        rows0[r, pl.ds(c * 16, 16)] = zero
        return 0

    lax.fori_loop(0, KS * D // 16, zfill, 0)
    for k in range(RPT // KS):
        pltpu.sync_copy(rows0, acc_sh.at[pl.ds(sid * RPT + k * KS, KS), :])

    pltpu.sync_copy(src_hbm.at[pl.ds(wid * EPAD, KS)], sidxA)
    pltpu.sync_copy(dst_hbm.at[pl.ds(wid * EPAD, KS)], didxA)
    plsc.subcore_barrier()

    def ifire(j, sbuf, dbuf):
        base = wid * EPAD + j * KS
        pltpu.async_copy(src_hbm.at[pl.ds(base, KS)], sbuf, isem)
        pltpu.async_copy(dst_hbm.at[pl.ds(base, KS)], dbuf, isem)

    def iwait(j, sbuf, dbuf):
        base = wid * EPAD + j * KS
        pltpu.make_async_copy(src_hbm.at[pl.ds(base, KS)], sbuf, isem).wait()
        pltpu.make_async_copy(dst_hbm.at[pl.ds(base, KS)], dbuf, isem).wait()

    def work(sbuf, dbuf):
        pltpu.async_copy(xs_hbm.at[sbuf], rows0, gsem0).wait()
        pltpu.sync_copy(rows0, acc_sh.at[dbuf], add=True)

    def pair(t, _):
        j = 2 * t
        jn2 = jnp.minimum(j + 2, NCHS - 1)
        ifire(j + 1, sidxB, didxB)
        work(sidxA, didxA)
        iwait(j + 1, sidxB, didxB)
        ifire(jn2, sidxA, didxA)
        work(sidxB, didxB)
        iwait(jn2, sidxA, didxA)
        return 0

    lax.fori_loop(0, NCHS // 2, pair, 0)
    plsc.subcore_barrier()

    pltpu.sync_copy(acc_sh.at[pl.ds(sid * RPT, RPT), :],
                    out_hbm.at[cid, pl.ds(sid * RPT, RPT), :])


RB = 400
GRID = N // RB


def _mm1_body(x_ref, w_ref, degA_ref, degB_ref, xs_ref, dis_ref):
    dis = lax.rsqrt(degA_ref[...] + degB_ref[...] + 1.0)
    xs_ref[...] = jnp.dot(x_ref[...], w_ref[...],
                          preferred_element_type=jnp.float32) * dis
    dis_ref[...] = dis


def _mm1(x, W1, degA, degB):
    return pl.pallas_call(
        _mm1_body,
        grid=(GRID,),
        in_specs=[
            pl.BlockSpec((RB, D), lambda i: (i, 0)),
            pl.BlockSpec((D, D), lambda i: (0, 0)),
            pl.BlockSpec((RB, 1), lambda i: (i, 0)),
            pl.BlockSpec((RB, 1), lambda i: (i, 0)),
        ],
        out_specs=[
            pl.BlockSpec((RB, D), lambda i: (i, 0)),
            pl.BlockSpec((RB, 1), lambda i: (i, 0)),
        ],
        out_shape=[
            jax.ShapeDtypeStruct((N, D), jnp.float32),
            jax.ShapeDtypeStruct((N, 1), jnp.float32),
        ],
    )(x, W1, degA, degB)


def _mm2_body(aA_ref, aB_ref, xs_ref, dis_ref, b_ref, w_ref, out_ref):
    dis = dis_ref[...]
    h = (aA_ref[...] + aB_ref[...] + xs_ref[...]) * dis + b_ref[...]
    h = jnp.maximum(h, 0.0)
    out_ref[...] = jnp.dot(h, w_ref[...],
                           preferred_element_type=jnp.float32) * dis


def _mm2(accA, accB, xs, dis, b, W2):
    return pl.pallas_call(
        _mm2_body,
        grid=(GRID,),
        in_specs=[
            pl.BlockSpec((RB, D), lambda i: (i, 0)),
            pl.BlockSpec((RB, D), lambda i: (i, 0)),
            pl.BlockSpec((RB, D), lambda i: (i, 0)),
            pl.BlockSpec((RB, 1), lambda i: (i, 0)),
            pl.BlockSpec((1, D), lambda i: (0, 0)),
            pl.BlockSpec((D, D), lambda i: (0, 0)),
        ],
        out_specs=pl.BlockSpec((RB, D), lambda i: (i, 0)),
        out_shape=jax.ShapeDtypeStruct((N, D), jnp.float32),
    )(accA, accB, xs, dis, b, W2)


def _fin_body(aA_ref, aB_ref, xs_ref, dis_ref, b_ref, out_ref):
    out_ref[...] = ((aA_ref[...] + aB_ref[...] + xs_ref[...]) * dis_ref[...]
                    + b_ref[...])


def _fin(accA, accB, xs, dis, b):
    return pl.pallas_call(
        _fin_body,
        grid=(GRID,),
        in_specs=[
            pl.BlockSpec((RB, D), lambda i: (i, 0)),
            pl.BlockSpec((RB, D), lambda i: (i, 0)),
            pl.BlockSpec((RB, D), lambda i: (i, 0)),
            pl.BlockSpec((RB, 1), lambda i: (i, 0)),
            pl.BlockSpec((1, D), lambda i: (0, 0)),
        ],
        out_specs=pl.BlockSpec((RB, D), lambda i: (i, 0)),
        out_shape=jax.ShapeDtypeStruct((N, D), jnp.float32),
    )(accA, accB, xs, dis, b)


def kernel(x, edge_index, W1, b1, W2, b2):
    srcp = jnp.pad(edge_index[0].reshape(NW, EPT), ((0, 0), (0, EPAD - EPT)))
    dstp = jnp.pad(edge_index[1].reshape(NW, EPT), ((0, 0), (0, EPAD - EPT)),
                   constant_values=N)
    src3 = srcp.reshape(NW * EPAD)
    dst3 = dstp.reshape(NW * EPAD)

    deg_p = _deg_kernel(dstp.reshape(NW, NCHD, KD))
    degA = deg_p[:NS].reshape(NPAD)[:N, None]
    degB = deg_p[NS:].reshape(NPAD)[:N, None]

    xs1, dis = _mm1(x, W1, degA, degB)
    acc1 = _scatter_kernel(xs1, src3, dst3)
    xs2 = _mm2(acc1[0, :N], acc1[1, :N], xs1, dis, b1[None, :], W2)
    acc2 = _scatter_kernel(xs2, src3, dst3)
    return _fin(acc2[0, :N], acc2[1, :N], xs2, dis, b2[None, :])


# speedup vs baseline: 2.0345x; 2.0345x over previous
import functools

import jax
import jax.numpy as jnp
from jax import lax
from jax.experimental import pallas as pl
from jax.experimental.pallas import tpu as pltpu
from jax.experimental.pallas import tpu_sc as plsc

N = 10000
E = 320000
D = 128

NC = 2
NS = 16
NW = NC * NS
EPT = E // NW
K = 128
KD = 128
NCHD = 80
KS = 80
NCHS = 125
EPAD = 10240
NPAD = 10240
WPT = NPAD // NS
RPT = NPAD // NS

_mesh = plsc.VectorSubcoreMesh(
    core_axis_name="c", subcore_axis_name="s", num_cores=NC, num_subcores=NS)


@functools.partial(
    pl.kernel,
    out_type=jax.ShapeDtypeStruct((NW, WPT), jnp.float32),
    mesh=_mesh,
    scratch_types=dict(
        deg_sh=pltpu.VMEM_SHARED((NPAD,), jnp.float32),
        zbuf=pltpu.VMEM((WPT,), jnp.float32),
        ones_v=pltpu.VMEM((KD,), jnp.float32),
        didx2=pltpu.VMEM((NCHD, KD), jnp.int32),
    ),
)
def _deg_kernel(dst_hbm, out_hbm, deg_sh, zbuf, ones_v, didx2):
    cid = lax.axis_index("c")
    sid = lax.axis_index("s")
    wid = sid * NC + cid

    one = jnp.ones((16,), jnp.float32)
    zero = jnp.zeros((16,), jnp.float32)

    def fill(i, _):
        zbuf[pl.ds(i * 16, 16)] = zero
        ones_v[pl.ds((i % (KD // 16)) * 16, 16)] = one
        return 0

    lax.fori_loop(0, WPT // 16, fill, 0)

    pltpu.sync_copy(dst_hbm.at[wid], didx2)
    pltpu.sync_copy(zbuf, deg_sh.at[pl.ds(sid * WPT, WPT)])
    plsc.subcore_barrier()

    def step(i, _):
        pltpu.sync_copy(ones_v, deg_sh.at[didx2.at[i]], add=True)
        return 0

    lax.fori_loop(0, NCHD, step, 0)
    plsc.subcore_barrier()

    pltpu.sync_copy(deg_sh.at[pl.ds(sid * WPT, WPT)],
                    out_hbm.at[cid * NS + sid])


@functools.partial(
    pl.kernel,
    out_type=jax.ShapeDtypeStruct((NC, NPAD, D), jnp.float32),
    mesh=_mesh,
    scratch_types=dict(
        acc_sh=pltpu.VMEM_SHARED((NPAD, D), jnp.float32),
        rows0=pltpu.VMEM((KS, D), jnp.float32),
        sidxA=pltpu.VMEM((KS,), jnp.int32),
        didxA=pltpu.VMEM((KS,), jnp.int32),
        sidxB=pltpu.VMEM((KS,), jnp.int32),
        didxB=pltpu.VMEM((KS,), jnp.int32),
        gsem0=pltpu.SemaphoreType.DMA,
        isem=pltpu.SemaphoreType.DMA,
    ),
)
def _scatter_kernel(xs_hbm, src_hbm, dst_hbm, out_hbm,
                    acc_sh, rows0, sidxA, didxA, sidxB, didxB, gsem0, isem):
    cid = lax.axis_index("c")
    sid = lax.axis_index("s")
    wid = sid * NC + cid

    zero = jnp.zeros((16,), jnp.float32)

    def zfill(i, _):
        r = i // 8
        c = i - r * 8
        rows0[r, pl.ds(c * 16, 16)] = zero
        return 0

    lax.fori_loop(0, KS * D // 16, zfill, 0)
    for k in range(RPT // KS):
        pltpu.sync_copy(rows0, acc_sh.at[pl.ds(sid * RPT + k * KS, KS), :])

    pltpu.sync_copy(src_hbm.at[pl.ds(wid * EPT, KS)], sidxA)
    pltpu.sync_copy(dst_hbm.at[pl.ds(wid * EPT, KS)], didxA)
    plsc.subcore_barrier()

    def ifire(j, sbuf, dbuf):
        base = wid * EPT + j * KS
        pltpu.async_copy(src_hbm.at[pl.ds(base, KS)], sbuf, isem)
        pltpu.async_copy(dst_hbm.at[pl.ds(base, KS)], dbuf, isem)

    def iwait(j, sbuf, dbuf):
        base = wid * EPT + j * KS
        pltpu.make_async_copy(src_hbm.at[pl.ds(base, KS)], sbuf, isem).wait()
        pltpu.make_async_copy(dst_hbm.at[pl.ds(base, KS)], dbuf, isem).wait()

    def work(sbuf, dbuf):
        pltpu.async_copy(xs_hbm.at[sbuf], rows0, gsem0).wait()
        pltpu.sync_copy(rows0, acc_sh.at[dbuf], add=True)

    def pair(t, _):
        j = 2 * t
        jn2 = jnp.minimum(j + 2, NCHS - 1)
        ifire(j + 1, sidxB, didxB)
        work(sidxA, didxA)
        iwait(j + 1, sidxB, didxB)
        ifire(jn2, sidxA, didxA)
        work(sidxB, didxB)
        iwait(jn2, sidxA, didxA)
        return 0

    lax.fori_loop(0, NCHS // 2, pair, 0)
    work(sidxA, didxA)
    plsc.subcore_barrier()

    pltpu.sync_copy(acc_sh.at[pl.ds(sid * RPT, RPT), :],
                    out_hbm.at[cid, pl.ds(sid * RPT, RPT), :])


RB = 400
GRID = N // RB


def _mm1_body(x_ref, w_ref, degA_ref, degB_ref, xs_ref, dis_ref):
    dis = lax.rsqrt(degA_ref[...] + degB_ref[...] + 1.0)
    xs_ref[...] = jnp.dot(x_ref[...], w_ref[...],
                          preferred_element_type=jnp.float32) * dis
    dis_ref[...] = dis


def _mm1(x, W1, degA, degB):
    return pl.pallas_call(
        _mm1_body,
        grid=(GRID,),
        in_specs=[
            pl.BlockSpec((RB, D), lambda i: (i, 0)),
            pl.BlockSpec((D, D), lambda i: (0, 0)),
            pl.BlockSpec((RB, 1), lambda i: (i, 0)),
            pl.BlockSpec((RB, 1), lambda i: (i, 0)),
        ],
        out_specs=[
            pl.BlockSpec((RB, D), lambda i: (i, 0)),
            pl.BlockSpec((RB, 1), lambda i: (i, 0)),
        ],
        out_shape=[
            jax.ShapeDtypeStruct((N, D), jnp.float32),
            jax.ShapeDtypeStruct((N, 1), jnp.float32),
        ],
    )(x, W1, degA, degB)


def _mm2_body(aA_ref, aB_ref, xs_ref, dis_ref, b_ref, w_ref, out_ref):
    dis = dis_ref[...]
    h = (aA_ref[...] + aB_ref[...] + xs_ref[...]) * dis + b_ref[...]
    h = jnp.maximum(h, 0.0)
    out_ref[...] = jnp.dot(h, w_ref[...],
                           preferred_element_type=jnp.float32) * dis


def _mm2(accA, accB, xs, dis, b, W2):
    return pl.pallas_call(
        _mm2_body,
        grid=(GRID,),
        in_specs=[
            pl.BlockSpec((RB, D), lambda i: (i, 0)),
            pl.BlockSpec((RB, D), lambda i: (i, 0)),
            pl.BlockSpec((RB, D), lambda i: (i, 0)),
            pl.BlockSpec((RB, 1), lambda i: (i, 0)),
            pl.BlockSpec((1, D), lambda i: (0, 0)),
            pl.BlockSpec((D, D), lambda i: (0, 0)),
        ],
        out_specs=pl.BlockSpec((RB, D), lambda i: (i, 0)),
        out_shape=jax.ShapeDtypeStruct((N, D), jnp.float32),
    )(accA, accB, xs, dis, b, W2)


def _fin_body(aA_ref, aB_ref, xs_ref, dis_ref, b_ref, out_ref):
    out_ref[...] = ((aA_ref[...] + aB_ref[...] + xs_ref[...]) * dis_ref[...]
                    + b_ref[...])


def _fin(accA, accB, xs, dis, b):
    return pl.pallas_call(
        _fin_body,
        grid=(GRID,),
        in_specs=[
            pl.BlockSpec((RB, D), lambda i: (i, 0)),
            pl.BlockSpec((RB, D), lambda i: (i, 0)),
            pl.BlockSpec((RB, D), lambda i: (i, 0)),
            pl.BlockSpec((RB, 1), lambda i: (i, 0)),
            pl.BlockSpec((1, D), lambda i: (0, 0)),
        ],
        out_specs=pl.BlockSpec((RB, D), lambda i: (i, 0)),
        out_shape=jax.ShapeDtypeStruct((N, D), jnp.float32),
    )(accA, accB, xs, dis, b)


def kernel(x, edge_index, W1, b1, W2, b2):
    srcp = jnp.pad(edge_index[0].reshape(NW, EPT), ((0, 0), (0, EPAD - EPT)))
    dstp = jnp.pad(edge_index[1].reshape(NW, EPT), ((0, 0), (0, EPAD - EPT)),
                   constant_values=N)
    src3 = edge_index[0]
    dst3 = edge_index[1]

    deg_p = _deg_kernel(dstp.reshape(NW, NCHD, KD))
    degA = deg_p[:NS].reshape(NPAD)[:N, None]
    degB = deg_p[NS:].reshape(NPAD)[:N, None]

    xs1, dis = _mm1(x, W1, degA, degB)
    acc1 = _scatter_kernel(xs1, src3, dst3)
    xs2 = _mm2(acc1[0, :N], acc1[1, :N], xs1, dis, b1[None, :], W2)
    acc2 = _scatter_kernel(xs2, src3, dst3)
    return _fin(acc2[0, :N], acc2[1, :N], xs2, dis, b2[None, :])


# speedup vs baseline: 2.6261x; 1.2908x over previous
import functools

import jax
import jax.numpy as jnp
from jax import lax
from jax.experimental import pallas as pl
from jax.experimental.pallas import tpu as pltpu
from jax.experimental.pallas import tpu_sc as plsc

N = 10000
E = 320000
D = 128

NC = 2
NS = 16
NW = NC * NS
EPT = E // NW
K = 128
KD = 128
NCHD = 80
KS = 80
NCHS = 125
EPAD = 10240
NPAD = 10240
WPT = NPAD // NS
RPT = NPAD // NS

_mesh = plsc.VectorSubcoreMesh(
    core_axis_name="c", subcore_axis_name="s", num_cores=NC, num_subcores=NS)


@functools.partial(
    pl.kernel,
    out_type=jax.ShapeDtypeStruct((NW, WPT), jnp.float32),
    mesh=_mesh,
    scratch_types=dict(
        deg_sh=pltpu.VMEM_SHARED((NPAD,), jnp.float32),
        zbuf=pltpu.VMEM((WPT,), jnp.float32),
        ones_v=pltpu.VMEM((KD,), jnp.float32),
        didx2=pltpu.VMEM((NCHD, KD), jnp.int32),
    ),
)
def _deg_kernel(dst_hbm, out_hbm, deg_sh, zbuf, ones_v, didx2):
    cid = lax.axis_index("c")
    sid = lax.axis_index("s")
    wid = sid * NC + cid

    one = jnp.ones((16,), jnp.float32)
    zero = jnp.zeros((16,), jnp.float32)

    def fill(i, _):
        zbuf[pl.ds(i * 16, 16)] = zero
        ones_v[pl.ds((i % (KD // 16)) * 16, 16)] = one
        return 0

    lax.fori_loop(0, WPT // 16, fill, 0)

    pltpu.sync_copy(dst_hbm.at[wid], didx2)
    pltpu.sync_copy(zbuf, deg_sh.at[pl.ds(sid * WPT, WPT)])
    plsc.subcore_barrier()

    def step(i, _):
        pltpu.sync_copy(ones_v, deg_sh.at[didx2.at[i]], add=True)
        return 0

    lax.fori_loop(0, NCHD, step, 0)
    plsc.subcore_barrier()

    pltpu.sync_copy(deg_sh.at[pl.ds(sid * WPT, WPT)],
                    out_hbm.at[cid * NS + sid])


@functools.partial(
    pl.kernel,
    out_type=jax.ShapeDtypeStruct((NC, NPAD, D), jnp.float32),
    mesh=_mesh,
    scratch_types=dict(
        acc_sh=pltpu.VMEM_SHARED((NPAD, D), jnp.float32),
        rows0=pltpu.VMEM((KS, D), jnp.float32),
        rows1=pltpu.VMEM((KS, D), jnp.float32),
        sidxA=pltpu.VMEM((KS,), jnp.int32),
        didxA=pltpu.VMEM((KS,), jnp.int32),
        sidxB=pltpu.VMEM((KS,), jnp.int32),
        didxB=pltpu.VMEM((KS,), jnp.int32),
        gsem0=pltpu.SemaphoreType.DMA,
        gsem1=pltpu.SemaphoreType.DMA,
        isem=pltpu.SemaphoreType.DMA,
    ),
)
def _scatter_kernel(xs_hbm, src_hbm, dst_hbm, out_hbm,
                    acc_sh, rows0, rows1, sidxA, didxA, sidxB, didxB,
                    gsem0, gsem1, isem):
    cid = lax.axis_index("c")
    sid = lax.axis_index("s")
    wid = sid * NC + cid

    zero = jnp.zeros((16,), jnp.float32)

    def zfill(i, _):
        r = i // 8
        c = i - r * 8
        rows0[r, pl.ds(c * 16, 16)] = zero
        return 0

    lax.fori_loop(0, KS * D // 16, zfill, 0)
    for k in range(RPT // KS):
        pltpu.sync_copy(rows0, acc_sh.at[pl.ds(sid * RPT + k * KS, KS), :])

    pltpu.sync_copy(src_hbm.at[pl.ds(wid * EPT, KS)], sidxA)
    pltpu.sync_copy(dst_hbm.at[pl.ds(wid * EPT, KS)], didxA)
    plsc.subcore_barrier()

    def ifire(j, sbuf, dbuf):
        base = wid * EPT + j * KS
        pltpu.async_copy(src_hbm.at[pl.ds(base, KS)], sbuf, isem)
        pltpu.async_copy(dst_hbm.at[pl.ds(base, KS)], dbuf, isem)

    def iwait(j, sbuf, dbuf):
        base = wid * EPT + j * KS
        pltpu.make_async_copy(src_hbm.at[pl.ds(base, KS)], sbuf, isem).wait()
        pltpu.make_async_copy(dst_hbm.at[pl.ds(base, KS)], dbuf, isem).wait()

    def gfire(sbuf, buf, sem):
        pltpu.async_copy(xs_hbm.at[sbuf], buf, sem)

    def gwait(sbuf, buf, sem):
        pltpu.make_async_copy(xs_hbm.at[sbuf], buf, sem).wait()

    def scat(dbuf, buf):
        pltpu.sync_copy(buf, acc_sh.at[dbuf], add=True)

    gfire(sidxA, rows0, gsem0)
    ifire(1, sidxB, didxB)

    def pair(t, _):
        j = 2 * t
        iwait(j + 1, sidxB, didxB)
        gfire(sidxB, rows1, gsem1)
        gwait(sidxA, rows0, gsem0)
        scat(didxA, rows0)
        ifire(j + 2, sidxA, didxA)
        iwait(j + 2, sidxA, didxA)
        gfire(sidxA, rows0, gsem0)
        gwait(sidxB, rows1, gsem1)
        scat(didxB, rows1)
        jn3 = jnp.minimum(j + 3, NCHS - 1)
        ifire(jn3, sidxB, didxB)
        return 0

    lax.fori_loop(0, NCHS // 2, pair, 0)
    gwait(sidxA, rows0, gsem0)
    scat(didxA, rows0)
    iwait(NCHS - 1, sidxB, didxB)
    plsc.subcore_barrier()

    pltpu.sync_copy(acc_sh.at[pl.ds(sid * RPT, RPT), :],
                    out_hbm.at[cid, pl.ds(sid * RPT, RPT), :])


RB = 400
GRID = N // RB


def _mm1_body(x_ref, w_ref, degA_ref, degB_ref, xs_ref, dis_ref):
    dis = lax.rsqrt(degA_ref[...] + degB_ref[...] + 1.0)
    xs_ref[...] = jnp.dot(x_ref[...], w_ref[...],
                          preferred_element_type=jnp.float32) * dis
    dis_ref[...] = dis


def _mm1(x, W1, degA, degB):
    return pl.pallas_call(
        _mm1_body,
        grid=(GRID,),
        in_specs=[
            pl.BlockSpec((RB, D), lambda i: (i, 0)),
            pl.BlockSpec((D, D), lambda i: (0, 0)),
            pl.BlockSpec((RB, 1), lambda i: (i, 0)),
            pl.BlockSpec((RB, 1), lambda i: (i, 0)),
        ],
        out_specs=[
            pl.BlockSpec((RB, D), lambda i: (i, 0)),
            pl.BlockSpec((RB, 1), lambda i: (i, 0)),
        ],
        out_shape=[
            jax.ShapeDtypeStruct((N, D), jnp.float32),
            jax.ShapeDtypeStruct((N, 1), jnp.float32),
        ],
    )(x, W1, degA, degB)


def _mm2_body(aA_ref, aB_ref, xs_ref, dis_ref, b_ref, w_ref, out_ref):
    dis = dis_ref[...]
    h = (aA_ref[...] + aB_ref[...] + xs_ref[...]) * dis + b_ref[...]
    h = jnp.maximum(h, 0.0)
    out_ref[...] = jnp.dot(h, w_ref[...],
                           preferred_element_type=jnp.float32) * dis


def _mm2(accA, accB, xs, dis, b, W2):
    return pl.pallas_call(
        _mm2_body,
        grid=(GRID,),
        in_specs=[
            pl.BlockSpec((RB, D), lambda i: (i, 0)),
            pl.BlockSpec((RB, D), lambda i: (i, 0)),
            pl.BlockSpec((RB, D), lambda i: (i, 0)),
            pl.BlockSpec((RB, 1), lambda i: (i, 0)),
            pl.BlockSpec((1, D), lambda i: (0, 0)),
            pl.BlockSpec((D, D), lambda i: (0, 0)),
        ],
        out_specs=pl.BlockSpec((RB, D), lambda i: (i, 0)),
        out_shape=jax.ShapeDtypeStruct((N, D), jnp.float32),
    )(accA, accB, xs, dis, b, W2)


def _fin_body(aA_ref, aB_ref, xs_ref, dis_ref, b_ref, out_ref):
    out_ref[...] = ((aA_ref[...] + aB_ref[...] + xs_ref[...]) * dis_ref[...]
                    + b_ref[...])


def _fin(accA, accB, xs, dis, b):
    return pl.pallas_call(
        _fin_body,
        grid=(GRID,),
        in_specs=[
            pl.BlockSpec((RB, D), lambda i: (i, 0)),
            pl.BlockSpec((RB, D), lambda i: (i, 0)),
            pl.BlockSpec((RB, D), lambda i: (i, 0)),
            pl.BlockSpec((RB, 1), lambda i: (i, 0)),
            pl.BlockSpec((1, D), lambda i: (0, 0)),
        ],
        out_specs=pl.BlockSpec((RB, D), lambda i: (i, 0)),
        out_shape=jax.ShapeDtypeStruct((N, D), jnp.float32),
    )(accA, accB, xs, dis, b)


def kernel(x, edge_index, W1, b1, W2, b2):
    srcp = jnp.pad(edge_index[0].reshape(NW, EPT), ((0, 0), (0, EPAD - EPT)))
    dstp = jnp.pad(edge_index[1].reshape(NW, EPT), ((0, 0), (0, EPAD - EPT)),
                   constant_values=N)
    src3 = edge_index[0]
    dst3 = edge_index[1]

    deg_p = _deg_kernel(dstp.reshape(NW, NCHD, KD))
    degA = deg_p[:NS].reshape(NPAD)[:N, None]
    degB = deg_p[NS:].reshape(NPAD)[:N, None]

    xs1, dis = _mm1(x, W1, degA, degB)
    acc1 = _scatter_kernel(xs1, src3, dst3)
    xs2 = _mm2(acc1[0, :N], acc1[1, :N], xs1, dis, b1[None, :], W2)
    acc2 = _scatter_kernel(xs2, src3, dst3)
    return _fin(acc2[0, :N], acc2[1, :N], xs2, dis, b2[None, :])
